# baseline (speedup 1.0000x reference)
import functools

import jax
import jax.numpy as jnp
from jax import lax
from jax.experimental import pallas as pl
from jax.experimental.pallas import tpu as pltpu
from jax.experimental.pallas import tpu_sc as plsc

SEQ_LEN = 200
EMBED = 64
VOCAB = 100000
CHUNK = 128
NUM_CORES = 2
NUM_SUBCORES = 16
NUM_WORKERS = NUM_CORES * NUM_SUBCORES
NBUF = 4


def _sc_body(
    idx_hbm, tok_hbm, pos_hbm, out_hbm, pos2_v, idx_v, rows, obuf, gsems, ssems
):
    wid = lax.axis_index("s") * NUM_CORES + lax.axis_index("c")
    n_chunks = idx_hbm.shape[0] // NUM_WORKERS
    chunk0 = wid * n_chunks

    pltpu.sync_copy(idx_hbm.at[pl.ds(chunk0, n_chunks)], idx_v)
    pltpu.sync_copy(pos_hbm, pos2_v.at[pl.ds(0, SEQ_LEN)])
    pltpu.sync_copy(pos_hbm, pos2_v.at[pl.ds(SEQ_LEN, SEQ_LEN)])

    def fire_gather(c, b):
        pltpu.async_copy(tok_hbm.at[idx_v.at[c]], rows.at[b], gsems.at[b])

    def drain_gather(b):
        pltpu.make_async_copy(
            tok_hbm.at[pl.ds(0, CHUNK)], rows.at[b], gsems.at[b]
        ).wait()

    def fire_scatter(c, b):
        pltpu.async_copy(
            obuf.at[b],
            out_hbm.at[pl.ds((chunk0 + c) * (CHUNK // 2), CHUNK // 2)],
            ssems.at[b],
        )

    def drain_scatter(b):
        pltpu.make_async_copy(
            obuf.at[b], out_hbm.at[pl.ds(0, CHUNK // 2)], ssems.at[b]
        ).wait()

    def add_pos(c, b):
        s0 = lax.rem(c * CHUNK, SEQ_LEN)

        def pair_body(p, carry):
            r = 2 * p
            for h in range(2):
                for e in range(EMBED // 16):
                    sl = pl.ds(e * 16, 16)
                    osl = pl.ds(h * EMBED + e * 16, 16)
                    obuf[b, p, osl] = rows[b, r + h, sl] + pos2_v[s0 + r + h, sl]
            return carry

        lax.fori_loop(0, CHUNK // 2, pair_body, 0)

    for b in range(NBUF):
        fire_gather(b, b)

    n_iters = n_chunks // NBUF

    def body(i, carry):
        for j in range(NBUF):
            c = i * NBUF + j
            drain_gather(j)

            @pl.when(i > 0)
            def _():
                drain_scatter(j)

            add_pos(c, j)
            fire_scatter(c, j)

            @pl.when(i < n_iters - 1)
            def _():
                fire_gather(c + NBUF, j)

        return carry

    lax.fori_loop(0, n_iters, body, 0)

    for b in range(NBUF):
        drain_scatter(b)


def kernel(inputs, token_table, position_table):
    batch = inputs.shape[0]
    idx = inputs.astype(jnp.int32).reshape(batch * SEQ_LEN // CHUNK, CHUNK)

    mesh = plsc.VectorSubcoreMesh(core_axis_name="c", subcore_axis_name="s")
    k = functools.partial(
        pl.kernel,
        out_type=jax.ShapeDtypeStruct((batch * SEQ_LEN // 2, 2 * EMBED), jnp.float32),
        mesh=mesh,
        compiler_params=pltpu.CompilerParams(use_tc_tiling_on_sc=False),
        scratch_types=[
            pltpu.VMEM((2 * SEQ_LEN, EMBED), jnp.float32),
            pltpu.VMEM((batch * SEQ_LEN // CHUNK // NUM_WORKERS, CHUNK), jnp.int32),
            pltpu.VMEM((NBUF, CHUNK, EMBED), jnp.float32),
            pltpu.VMEM((NBUF, CHUNK // 2, 2 * EMBED), jnp.float32),
            pltpu.SemaphoreType.DMA((NBUF,)),
            pltpu.SemaphoreType.DMA((NBUF,)),
        ],
    )(_sc_body)
    out = k(idx, token_table, position_table)
    return out.reshape(batch, SEQ_LEN, EMBED)
